# Initial kernel scaffold; baseline (speedup 1.0000x reference)
import jax
import jax.numpy as jnp
from jax.experimental import pallas as pl


def kernel(pe, indexes):
    raise NotImplementedError("write your pallas kernel here")


# speedup vs baseline: 2.1773x; 2.1773x over previous
import functools

import jax
import jax.numpy as jnp
from jax import lax
from jax.experimental import pallas as pl
from jax.experimental.pallas import tpu as pltpu
from jax.experimental.pallas import tpu_sc as plsc

D_MODEL = 1024
NUM_CORES = 2
NUM_SUBCORES = 16
NUM_WORKERS = NUM_CORES * NUM_SUBCORES
CHUNK = 64


@functools.partial(jax.jit, static_argnames=())
def _gather_rows(pe, idx_flat):
    n = idx_flat.shape[0]
    b_per_w = n // NUM_WORKERS
    n_chunks = b_per_w // CHUNK
    mesh = plsc.VectorSubcoreMesh(core_axis_name="c", subcore_axis_name="s")

    @functools.partial(
        pl.kernel,
        mesh=mesh,
        out_type=jax.ShapeDtypeStruct((n, D_MODEL), jnp.float32),
        scratch_types=[
            pltpu.VMEM((b_per_w,), jnp.int32),
            pltpu.VMEM((CHUNK, D_MODEL), jnp.float32),
            pltpu.SemaphoreType.DMA,
        ],
    )
    def k(table_hbm, idx_hbm, out_hbm, idx_v, rows_v, sem):
        wid = lax.axis_index("s") * NUM_CORES + lax.axis_index("c")
        base = wid * b_per_w
        pltpu.sync_copy(idx_hbm.at[pl.ds(base, b_per_w)], idx_v)

        @pl.loop(0, n_chunks)
        def _(ci):
            off = ci * CHUNK
            pltpu.async_copy(
                table_hbm.at[idx_v.at[pl.ds(off, CHUNK)]], rows_v, sem
            ).wait()
            pltpu.sync_copy(rows_v, out_hbm.at[pl.ds(base + off, CHUNK)])

    return k(pe, idx_flat)


def kernel(pe, indexes):
    b, s = indexes.shape
    idx_flat = indexes.astype(jnp.int32).reshape(b * s)
    out = _gather_rows(pe, idx_flat)
    return out.reshape(b, s, D_MODEL)


# speedup vs baseline: 2.3763x; 1.0914x over previous
import functools

import jax
import jax.numpy as jnp
from jax import lax
from jax.experimental import pallas as pl
from jax.experimental.pallas import tpu as pltpu
from jax.experimental.pallas import tpu_sc as plsc

D_MODEL = 1024
NUM_CORES = 2
NUM_SUBCORES = 16
NUM_WORKERS = NUM_CORES * NUM_SUBCORES
CHUNK = 32


@functools.partial(jax.jit, static_argnames=())
def _gather_rows(pe, idx_flat):
    n = idx_flat.shape[0]
    b_per_w = n // NUM_WORKERS
    n_chunks = b_per_w // CHUNK
    mesh = plsc.VectorSubcoreMesh(core_axis_name="c", subcore_axis_name="s")

    @functools.partial(
        pl.kernel,
        mesh=mesh,
        out_type=jax.ShapeDtypeStruct((n, D_MODEL), jnp.float32),
        scratch_types=[
            pltpu.VMEM((b_per_w,), jnp.int32),
            pltpu.VMEM((CHUNK, D_MODEL), jnp.float32),
            pltpu.VMEM((CHUNK, D_MODEL), jnp.float32),
            pltpu.SemaphoreType.DMA,
            pltpu.SemaphoreType.DMA,
        ],
    )
    def k(table_hbm, idx_hbm, out_hbm, idx_v, rows0, rows1, sem0, sem1):
        wid = lax.axis_index("s") * NUM_CORES + lax.axis_index("c")
        base = wid * b_per_w
        pltpu.sync_copy(idx_hbm.at[pl.ds(base, b_per_w)], idx_v)

        def gather_start(ci, rows, sem):
            pltpu.async_copy(
                table_hbm.at[idx_v.at[pl.ds(ci * CHUNK, CHUNK)]], rows, sem
            )

        def gather_wait(rows, sem):
            pltpu.make_async_copy(
                table_hbm.at[idx_v.at[pl.ds(0, CHUNK)]], rows, sem
            ).wait()

        gather_start(0, rows0, sem0)
        gather_start(1, rows1, sem1)

        @pl.loop(0, n_chunks, step=2)
        def _(ci):
            for b, rows, sem in ((0, rows0, sem0), (1, rows1, sem1)):
                gather_wait(rows, sem)
                pltpu.sync_copy(
                    rows, out_hbm.at[pl.ds(base + (ci + b) * CHUNK, CHUNK)]
                )

                @pl.when(ci + b + 2 < n_chunks)
                def _():
                    gather_start(ci + b + 2, rows, sem)

    return k(pe, idx_flat)


def kernel(pe, indexes):
    b, s = indexes.shape
    idx_flat = indexes.astype(jnp.int32).reshape(b * s)
    out = _gather_rows(pe, idx_flat)
    return out.reshape(b, s, D_MODEL)


# speedup vs baseline: 2.3786x; 1.0010x over previous
import functools

import jax
import jax.numpy as jnp
from jax import lax
from jax.experimental import pallas as pl
from jax.experimental.pallas import tpu as pltpu
from jax.experimental.pallas import tpu_sc as plsc

D_MODEL = 1024
NUM_CORES = 2
NUM_SUBCORES = 16
NUM_WORKERS = NUM_CORES * NUM_SUBCORES
CHUNK = 16
NBUF = 4


@functools.partial(jax.jit, static_argnames=())
def _gather_rows(pe, idx_flat):
    n = idx_flat.shape[0]
    b_per_w = n // NUM_WORKERS
    n_chunks = b_per_w // CHUNK
    mesh = plsc.VectorSubcoreMesh(core_axis_name="c", subcore_axis_name="s")

    @functools.partial(
        pl.kernel,
        mesh=mesh,
        out_type=jax.ShapeDtypeStruct((n, D_MODEL), jnp.float32),
        scratch_types=[
            pltpu.VMEM((b_per_w,), jnp.int32),
        ]
        + [pltpu.VMEM((CHUNK, D_MODEL), jnp.float32)] * NBUF
        + [pltpu.SemaphoreType.DMA] * (2 * NBUF),
    )
    def k(table_hbm, idx_hbm, out_hbm, idx_v, *bufs_and_sems):
        bufs = bufs_and_sems[:NBUF]
        gsems = bufs_and_sems[NBUF : 2 * NBUF]
        wsems = bufs_and_sems[2 * NBUF :]
        wid = lax.axis_index("s") * NUM_CORES + lax.axis_index("c")
        base = wid * b_per_w
        pltpu.sync_copy(idx_hbm.at[pl.ds(base, b_per_w)], idx_v)

        def gather_start(ci, s):
            pltpu.async_copy(
                table_hbm.at[idx_v.at[pl.ds(ci * CHUNK, CHUNK)]], bufs[s], gsems[s]
            )

        def gather_wait(s):
            pltpu.make_async_copy(
                table_hbm.at[idx_v.at[pl.ds(0, CHUNK)]], bufs[s], gsems[s]
            ).wait()

        def write_start(ci, s):
            pltpu.async_copy(
                bufs[s], out_hbm.at[pl.ds(base + ci * CHUNK, CHUNK)], wsems[s]
            )

        def write_drain(s):
            pltpu.make_async_copy(
                bufs[s], out_hbm.at[pl.ds(base, CHUNK)], wsems[s]
            ).wait()

        gather_start(0, 0)
        gather_start(1, 1)

        @pl.loop(0, n_chunks, step=NBUF)
        def _(j):
            for s in range(NBUF):
                c = j + s
                s2 = (s + 2) % NBUF
                gather_wait(s)
                write_start(c, s)

                @pl.when(c + 2 < n_chunks)
                def _(c=c, s2=s2):
                    @pl.when(c >= 2)
                    def _():
                        write_drain(s2)

                    gather_start(c + 2, s2)

        for s in range(NBUF):
            write_drain(s)

    return k(pe, idx_flat)


def kernel(pe, indexes):
    b, s = indexes.shape
    idx_flat = indexes.astype(jnp.int32).reshape(b * s)
    out = _gather_rows(pe, idx_flat)
    return out.reshape(b, s, D_MODEL)
